# baseline (speedup 1.0000x reference)
import jax
import jax.numpy as jnp
from jax.experimental import pallas as pl
from jax.experimental.pallas import tpu as pltpu


def _pick_bb(b, target):
    for t in range(target, 0, -1):
        if b % t == 0:
            return t
    return 1


def _up_matrix(n):
    p = jnp.arange(2 * n, dtype=jnp.float32)
    src = jnp.maximum(p * 0.5 - 0.25, 0.0)
    i0 = jnp.floor(src).astype(jnp.int32)
    i1 = jnp.minimum(i0 + 1, n - 1)
    lam = src - i0.astype(jnp.float32)
    cols = jnp.arange(n, dtype=jnp.int32)
    return ((cols[None, :] == i0[:, None]).astype(jnp.float32) * (1.0 - lam)[:, None]
            + (cols[None, :] == i1[:, None]).astype(jnp.float32) * lam[:, None])


def _make_conv_kernel(bb):
    def _conv_kernel(istyle_ref, wst_ref, bst_ref, wconv_ref, x_ref, o_ref):
        g = pl.program_id(0)
        wst = wst_ref[...]
        bst = bst_ref[...]
        wconv = wconv_ref[...]
        for i in range(bb):
            sty = istyle_ref[pl.ds(g * bb + i, 1), :]
            style = jnp.dot(sty, wst, preferred_element_type=jnp.float32) + bst
            w_mod = (wconv * (style + 1.0)).astype(jnp.bfloat16)
            o_ref[i] = jnp.dot(w_mod, x_ref[i], preferred_element_type=jnp.float32)
    return _conv_kernel


def _make_up_kernel(bb):
    def _residual_up_kernel(uh3_ref, uwt_ref, rgb_ref, prev_ref, o_ref):
        O, H, W = prev_ref.shape[1], prev_ref.shape[2], prev_ref.shape[3]
        uh3 = uh3_ref[...]
        uwt = uwt_ref[...]
        for i in range(bb):
            rows = (rgb_ref[i] + prev_ref[i]).reshape(O * H, W)
            t = jnp.dot(uh3, rows, preferred_element_type=jnp.float32)
            y = jnp.dot(t, uwt, preferred_element_type=jnp.float32)
            o_ref[i] = y.reshape(O, 2 * H, 2 * W)
    return _residual_up_kernel


def kernel(x, prev_rgb, istyle, style_w, style_b, conv_w):
    B, C, H, W = x.shape
    L = istyle.shape[1]
    O = conv_w.shape[0]
    HW = H * W
    itemsize = jnp.dtype(x.dtype).itemsize

    bb_conv = _pick_bb(B, 4)
    bb_up = _pick_bb(B, 8)

    x_flat = x.astype(jnp.bfloat16).reshape(B, C, HW)
    wst = jnp.transpose(style_w)
    bst = style_b.reshape(1, C)
    wconv = conv_w.reshape(O, C)

    conv_cost = pl.CostEstimate(
        flops=2 * B * L * C + 2 * B * O * C * HW,
        transcendentals=0,
        bytes_accessed=(B * O * HW + B * L) * itemsize + B * C * HW * 2
        + (L * C + C + O * C) * itemsize,
    )
    rgb_flat = pl.pallas_call(
        _make_conv_kernel(bb_conv),
        out_shape=jax.ShapeDtypeStruct((B, O, HW), x.dtype),
        grid_spec=pltpu.PrefetchScalarGridSpec(
            num_scalar_prefetch=0,
            grid=(B // bb_conv,),
            in_specs=[
                pl.BlockSpec((B, L), lambda g: (0, 0)),
                pl.BlockSpec((L, C), lambda g: (0, 0)),
                pl.BlockSpec((1, C), lambda g: (0, 0)),
                pl.BlockSpec((O, C), lambda g: (0, 0)),
                pl.BlockSpec((bb_conv, C, HW), lambda g: (g, 0, 0)),
            ],
            out_specs=pl.BlockSpec((bb_conv, O, HW), lambda g: (g, 0, 0)),
        ),
        compiler_params=pltpu.CompilerParams(dimension_semantics=("parallel",)),
        cost_estimate=conv_cost,
    )(istyle, wst, bst, wconv, x_flat)

    rgb4 = rgb_flat.reshape(B, O, H, W)

    uh = _up_matrix(H)
    uwt = jnp.transpose(_up_matrix(W))
    uh3 = jnp.zeros((O * 2 * H, O * H), dtype=jnp.float32)
    for o in range(O):
        uh3 = uh3.at[o * 2 * H:(o + 1) * 2 * H, o * H:(o + 1) * H].set(uh)

    up_cost = pl.CostEstimate(
        flops=2 * B * (O * 2 * H * O * H * W + O * 2 * H * W * 2 * W) + B * O * HW,
        transcendentals=0,
        bytes_accessed=(2 * B * O * HW + B * O * 4 * HW) * itemsize
        + (O * O * 2 * H * H + 2 * W * W) * itemsize,
    )
    return pl.pallas_call(
        _make_up_kernel(bb_up),
        out_shape=jax.ShapeDtypeStruct((B, O, 2 * H, 2 * W), x.dtype),
        grid_spec=pltpu.PrefetchScalarGridSpec(
            num_scalar_prefetch=0,
            grid=(B // bb_up,),
            in_specs=[
                pl.BlockSpec((O * 2 * H, O * H), lambda g: (0, 0)),
                pl.BlockSpec((W, 2 * W), lambda g: (0, 0)),
                pl.BlockSpec((bb_up, O, H, W), lambda g: (g, 0, 0, 0)),
                pl.BlockSpec((bb_up, O, H, W), lambda g: (g, 0, 0, 0)),
            ],
            out_specs=pl.BlockSpec((bb_up, O, 2 * H, 2 * W), lambda g: (g, 0, 0, 0)),
        ),
        compiler_params=pltpu.CompilerParams(dimension_semantics=("parallel",)),
        cost_estimate=up_cost,
    )(uh3, uwt, rgb4, prev_rgb)


# speedup vs baseline: 1.0628x; 1.0628x over previous
import jax
import jax.numpy as jnp
from jax.experimental import pallas as pl
from jax.experimental.pallas import tpu as pltpu

def _pick_bb(b, target):
    for t in range(target, 0, -1):
        if b % t == 0:
            return t
    return 1


def _up_matrix(n):
    p = jnp.arange(2 * n, dtype=jnp.float32)
    src = jnp.maximum(p * 0.5 - 0.25, 0.0)
    i0 = jnp.floor(src).astype(jnp.int32)
    i1 = jnp.minimum(i0 + 1, n - 1)
    lam = src - i0.astype(jnp.float32)
    cols = jnp.arange(n, dtype=jnp.int32)
    return ((cols[None, :] == i0[:, None]).astype(jnp.float32) * (1.0 - lam)[:, None]
            + (cols[None, :] == i1[:, None]).astype(jnp.float32) * lam[:, None])


def _make_conv_kernel(bb):
    def _conv_kernel(istyle_ref, wst_ref, bst_ref, wconv_ref, x_ref, o_ref):
        g = pl.program_id(0)
        wst = wst_ref[...]
        bst = bst_ref[...]
        wconv = wconv_ref[...]
        for i in range(bb):
            sty = istyle_ref[pl.ds(g * bb + i, 1), :]
            style = jnp.dot(sty, wst, preferred_element_type=jnp.float32) + bst
            w_mod = wconv * (style + 1.0)
            o_ref[i] = jnp.dot(w_mod, x_ref[i], preferred_element_type=jnp.float32)
    return _conv_kernel


def _make_up_kernel(bb):
    def _residual_up_kernel(uh3_ref, uwt_ref, rgb_ref, prev_ref, o_ref):
        O, H, W = prev_ref.shape[1], prev_ref.shape[2], prev_ref.shape[3]
        uh3 = uh3_ref[...]
        uwt = uwt_ref[...]
        for i in range(bb):
            rows = (rgb_ref[i] + prev_ref[i]).reshape(O * H, W)
            t = jnp.dot(uh3, rows, preferred_element_type=jnp.float32)
            y = jnp.dot(t, uwt, preferred_element_type=jnp.float32)
            o_ref[i] = y.reshape(O, 2 * H, 2 * W)
    return _residual_up_kernel


def kernel(x, prev_rgb, istyle, style_w, style_b, conv_w):
    B, C, H, W = x.shape
    L = istyle.shape[1]
    O = conv_w.shape[0]
    HW = H * W
    itemsize = jnp.dtype(x.dtype).itemsize

    bb_conv = _pick_bb(B, 4)
    bb_up = _pick_bb(B, 16)

    x_flat = x.reshape(B, C, HW)
    wst = jnp.transpose(style_w)
    bst = style_b.reshape(1, C)
    wconv = conv_w.reshape(O, C)

    conv_cost = pl.CostEstimate(
        flops=2 * B * L * C + 2 * B * O * C * HW,
        transcendentals=0,
        bytes_accessed=(B * C * HW + B * O * HW + B * L) * itemsize
        + (L * C + C + O * C) * itemsize,
    )
    rgb_flat = pl.pallas_call(
        _make_conv_kernel(bb_conv),
        out_shape=jax.ShapeDtypeStruct((B, O, HW), x.dtype),
        grid_spec=pltpu.PrefetchScalarGridSpec(
            num_scalar_prefetch=0,
            grid=(B // bb_conv,),
            in_specs=[
                pl.BlockSpec((B, L), lambda g: (0, 0)),
                pl.BlockSpec((L, C), lambda g: (0, 0)),
                pl.BlockSpec((1, C), lambda g: (0, 0)),
                pl.BlockSpec((O, C), lambda g: (0, 0)),
                pl.BlockSpec((bb_conv, C, HW), lambda g: (g, 0, 0)),
            ],
            out_specs=pl.BlockSpec((bb_conv, O, HW), lambda g: (g, 0, 0)),
        ),
        compiler_params=pltpu.CompilerParams(dimension_semantics=("parallel",)),
        cost_estimate=conv_cost,
    )(istyle, wst, bst, wconv, x_flat)

    rgb4 = rgb_flat.reshape(B, O, H, W)

    uh = _up_matrix(H)
    uwt = jnp.transpose(_up_matrix(W))
    uh3 = jnp.zeros((O * 2 * H, O * H), dtype=jnp.float32)
    for o in range(O):
        uh3 = uh3.at[o * 2 * H:(o + 1) * 2 * H, o * H:(o + 1) * H].set(uh)

    up_cost = pl.CostEstimate(
        flops=2 * B * (O * 2 * H * O * H * W + O * 2 * H * W * 2 * W) + B * O * HW,
        transcendentals=0,
        bytes_accessed=(2 * B * O * HW + B * O * 4 * HW) * itemsize
        + (O * O * 2 * H * H + 2 * W * W) * itemsize,
    )
    return pl.pallas_call(
        _make_up_kernel(bb_up),
        out_shape=jax.ShapeDtypeStruct((B, O, 2 * H, 2 * W), x.dtype),
        grid_spec=pltpu.PrefetchScalarGridSpec(
            num_scalar_prefetch=0,
            grid=(B // bb_up,),
            in_specs=[
                pl.BlockSpec((O * 2 * H, O * H), lambda g: (0, 0)),
                pl.BlockSpec((W, 2 * W), lambda g: (0, 0)),
                pl.BlockSpec((bb_up, O, H, W), lambda g: (g, 0, 0, 0)),
                pl.BlockSpec((bb_up, O, H, W), lambda g: (g, 0, 0, 0)),
            ],
            out_specs=pl.BlockSpec((bb_up, O, 2 * H, 2 * W), lambda g: (g, 0, 0, 0)),
        ),
        compiler_params=pltpu.CompilerParams(dimension_semantics=("parallel",)),
        cost_estimate=up_cost,
    )(uh3, uwt, rgb4, prev_rgb)


# speedup vs baseline: 1.0658x; 1.0028x over previous
import jax
import jax.numpy as jnp
from jax.experimental import pallas as pl
from jax.experimental.pallas import tpu as pltpu

def _pick_bb(b, target):
    for t in range(target, 0, -1):
        if b % t == 0:
            return t
    return 1


def _up_matrix(n):
    p = jnp.arange(2 * n, dtype=jnp.float32)
    src = jnp.maximum(p * 0.5 - 0.25, 0.0)
    i0 = jnp.floor(src).astype(jnp.int32)
    i1 = jnp.minimum(i0 + 1, n - 1)
    lam = src - i0.astype(jnp.float32)
    cols = jnp.arange(n, dtype=jnp.int32)
    return ((cols[None, :] == i0[:, None]).astype(jnp.float32) * (1.0 - lam)[:, None]
            + (cols[None, :] == i1[:, None]).astype(jnp.float32) * lam[:, None])


def _make_conv_kernel(bb):
    def _conv_kernel(istyle_ref, wst_ref, bst_ref, wconv_ref, xa_ref, xb_ref,
                     o_ref):
        g = pl.program_id(0)
        hw2 = xa_ref.shape[2]
        wst = wst_ref[...]
        bst = bst_ref[...]
        wconv = wconv_ref[...]
        for i in range(bb):
            sty = istyle_ref[pl.ds(g * bb + i, 1), :]
            style = jnp.dot(sty, wst, preferred_element_type=jnp.float32) + bst
            w_mod = wconv * (style + 1.0)
            o_ref[i, :, :hw2] = jnp.dot(w_mod, xa_ref[i],
                                        preferred_element_type=jnp.float32)
            o_ref[i, :, hw2:] = jnp.dot(w_mod, xb_ref[i],
                                        preferred_element_type=jnp.float32)
    return _conv_kernel


def _make_up_kernel(bb):
    def _residual_up_kernel(uh3_ref, uwt_ref, rgb_ref, prev_ref, o_ref):
        O, H, W = prev_ref.shape[1], prev_ref.shape[2], prev_ref.shape[3]
        uh3 = uh3_ref[...]
        uwt = uwt_ref[...]
        for i in range(bb):
            rows = (rgb_ref[i] + prev_ref[i]).reshape(O * H, W)
            t = jnp.dot(uh3, rows, preferred_element_type=jnp.float32)
            y = jnp.dot(t, uwt, preferred_element_type=jnp.float32)
            o_ref[i] = y.reshape(O, 2 * H, 2 * W)
    return _residual_up_kernel


def kernel(x, prev_rgb, istyle, style_w, style_b, conv_w):
    B, C, H, W = x.shape
    L = istyle.shape[1]
    O = conv_w.shape[0]
    HW = H * W
    itemsize = jnp.dtype(x.dtype).itemsize

    bb_conv = _pick_bb(B, 4)
    bb_up = _pick_bb(B, 8)

    x_flat = x.reshape(B, C, HW)
    wst = jnp.transpose(style_w)
    bst = style_b.reshape(1, C)
    wconv = conv_w.reshape(O, C)

    conv_cost = pl.CostEstimate(
        flops=2 * B * L * C + 2 * B * O * C * HW,
        transcendentals=0,
        bytes_accessed=(B * C * HW + B * O * HW + B * L) * itemsize
        + (L * C + C + O * C) * itemsize,
    )
    rgb_flat = pl.pallas_call(
        _make_conv_kernel(bb_conv),
        out_shape=jax.ShapeDtypeStruct((B, O, HW), x.dtype),
        grid_spec=pltpu.PrefetchScalarGridSpec(
            num_scalar_prefetch=0,
            grid=(B // bb_conv,),
            in_specs=[
                pl.BlockSpec((B, L), lambda g: (0, 0)),
                pl.BlockSpec((L, C), lambda g: (0, 0)),
                pl.BlockSpec((1, C), lambda g: (0, 0)),
                pl.BlockSpec((O, C), lambda g: (0, 0)),
                pl.BlockSpec((bb_conv, C, HW // 2), lambda g: (g, 0, 0)),
                pl.BlockSpec((bb_conv, C, HW // 2), lambda g: (g, 0, 1)),
            ],
            out_specs=pl.BlockSpec((bb_conv, O, HW), lambda g: (g, 0, 0)),
        ),
        compiler_params=pltpu.CompilerParams(dimension_semantics=("parallel",)),
        cost_estimate=conv_cost,
    )(istyle, wst, bst, wconv, x_flat, x_flat)

    rgb4 = rgb_flat.reshape(B, O, H, W)

    uh = _up_matrix(H)
    uwt = jnp.transpose(_up_matrix(W))
    uh3 = jnp.zeros((O * 2 * H, O * H), dtype=jnp.float32)
    for o in range(O):
        uh3 = uh3.at[o * 2 * H:(o + 1) * 2 * H, o * H:(o + 1) * H].set(uh)

    up_cost = pl.CostEstimate(
        flops=2 * B * (O * 2 * H * O * H * W + O * 2 * H * W * 2 * W) + B * O * HW,
        transcendentals=0,
        bytes_accessed=(2 * B * O * HW + B * O * 4 * HW) * itemsize
        + (O * O * 2 * H * H + 2 * W * W) * itemsize,
    )
    return pl.pallas_call(
        _make_up_kernel(bb_up),
        out_shape=jax.ShapeDtypeStruct((B, O, 2 * H, 2 * W), x.dtype),
        grid_spec=pltpu.PrefetchScalarGridSpec(
            num_scalar_prefetch=0,
            grid=(B // bb_up,),
            in_specs=[
                pl.BlockSpec((O * 2 * H, O * H), lambda g: (0, 0)),
                pl.BlockSpec((W, 2 * W), lambda g: (0, 0)),
                pl.BlockSpec((bb_up, O, H, W), lambda g: (g, 0, 0, 0)),
                pl.BlockSpec((bb_up, O, H, W), lambda g: (g, 0, 0, 0)),
            ],
            out_specs=pl.BlockSpec((bb_up, O, 2 * H, 2 * W), lambda g: (g, 0, 0, 0)),
        ),
        compiler_params=pltpu.CompilerParams(dimension_semantics=("parallel",)),
        cost_estimate=up_cost,
    )(uh3, uwt, rgb4, prev_rgb)


# speedup vs baseline: 1.1171x; 1.0481x over previous
import jax
import jax.numpy as jnp
from jax.experimental import pallas as pl
from jax.experimental.pallas import tpu as pltpu


def _pick_bb(b, target):
    for t in range(target, 0, -1):
        if b % t == 0:
            return t
    return 1


def _up_matrix(n):
    p = jnp.arange(2 * n, dtype=jnp.float32)
    src = jnp.maximum(p * 0.5 - 0.25, 0.0)
    i0 = jnp.floor(src).astype(jnp.int32)
    i1 = jnp.minimum(i0 + 1, n - 1)
    lam = src - i0.astype(jnp.float32)
    cols = jnp.arange(n, dtype=jnp.int32)
    return ((cols[None, :] == i0[:, None]).astype(jnp.float32) * (1.0 - lam)[:, None]
            + (cols[None, :] == i1[:, None]).astype(jnp.float32) * lam[:, None])


def _make_fused_kernel(bb):
    def _fused_kernel(istyle_ref, wst_ref, bst_ref, wconv_ref, uh3_ref,
                      uwt_ref, x_ref, prev_ref, o_ref, scr_ref):
        g = pl.program_id(0)
        O, H, W = prev_ref.shape[1], prev_ref.shape[2], prev_ref.shape[3]
        wst = wst_ref[...]
        bst = bst_ref[...]
        wconv = wconv_ref[...]
        uh3 = uh3_ref[...]
        uwt = uwt_ref[...]
        for i in range(bb):
            sty = istyle_ref[pl.ds(g * bb + i, 1), :]
            style = jnp.dot(sty, wst, preferred_element_type=jnp.float32) + bst
            w_mod = wconv * (style + 1.0)
            rgb = jnp.dot(w_mod, x_ref[i], preferred_element_type=jnp.float32)
            for h in range(H):
                scr_ref[:, h, :] = rgb[:, h * W:(h + 1) * W]
            rows = scr_ref[...].reshape(O * H, W) + prev_ref[i].reshape(O * H, W)
            t = jnp.dot(uh3, rows, preferred_element_type=jnp.float32)
            y = jnp.dot(t, uwt, preferred_element_type=jnp.float32)
            o_ref[i] = y.reshape(O, 2 * H, 2 * W)
    return _fused_kernel


def kernel(x, prev_rgb, istyle, style_w, style_b, conv_w):
    B, C, H, W = x.shape
    L = istyle.shape[1]
    O = conv_w.shape[0]
    HW = H * W
    itemsize = jnp.dtype(x.dtype).itemsize

    bb = _pick_bb(B, 4)

    x_flat = x.reshape(B, C, HW)
    wst = jnp.transpose(style_w)
    bst = style_b.reshape(1, C)
    wconv = conv_w.reshape(O, C)

    uh = _up_matrix(H)
    uwt = jnp.transpose(_up_matrix(W))
    uh3 = jnp.zeros((O * 2 * H, O * H), dtype=jnp.float32)
    for o in range(O):
        uh3 = uh3.at[o * 2 * H:(o + 1) * 2 * H, o * H:(o + 1) * H].set(uh)

    cost = pl.CostEstimate(
        flops=2 * B * L * C + 2 * B * O * C * HW
        + 2 * B * (O * 2 * H * O * H * W + O * 2 * H * W * 2 * W),
        transcendentals=0,
        bytes_accessed=(B * C * HW + B * O * HW + B * O * 4 * HW + B * L)
        * itemsize
        + (L * C + C + O * C + O * O * 2 * H * H + 2 * W * W) * itemsize,
    )
    return pl.pallas_call(
        _make_fused_kernel(bb),
        out_shape=jax.ShapeDtypeStruct((B, O, 2 * H, 2 * W), x.dtype),
        grid_spec=pltpu.PrefetchScalarGridSpec(
            num_scalar_prefetch=0,
            grid=(B // bb,),
            in_specs=[
                pl.BlockSpec((B, L), lambda g: (0, 0)),
                pl.BlockSpec((L, C), lambda g: (0, 0)),
                pl.BlockSpec((1, C), lambda g: (0, 0)),
                pl.BlockSpec((O, C), lambda g: (0, 0)),
                pl.BlockSpec((O * 2 * H, O * H), lambda g: (0, 0)),
                pl.BlockSpec((W, 2 * W), lambda g: (0, 0)),
                pl.BlockSpec((bb, C, HW), lambda g: (g, 0, 0)),
                pl.BlockSpec((bb, O, H, W), lambda g: (g, 0, 0, 0)),
            ],
            out_specs=pl.BlockSpec((bb, O, 2 * H, 2 * W), lambda g: (g, 0, 0, 0)),
            scratch_shapes=[pltpu.VMEM((O, H, W), jnp.float32)],
        ),
        compiler_params=pltpu.CompilerParams(dimension_semantics=("parallel",)),
        cost_estimate=cost,
    )(istyle, wst, bst, wconv, uh3, uwt, x_flat, prev_rgb)
